# baseline (speedup 1.0000x reference)
import functools

import jax
import jax.numpy as jnp
from jax import lax
from jax.experimental import pallas as pl
from jax.experimental.pallas import tpu as pltpu
from jax.experimental.pallas import tpu_sc as plsc

BATCH = 16384
DIM = 64
NUSERS = 1000000

_info = plsc.get_sparse_core_info()
NC, NS, L = _info.num_cores, _info.num_subcores, _info.num_lanes
NW = NC * NS
BPW = BATCH // NW
NBLK = (NUSERS + 127) // 128
NBPT = (NBLK + NW - 1) // NW
RING = 5
SR = 8
BIG = 2**31 - 1
CHU = 128
NVR = BATCH // L

_mesh = plsc.VectorSubcoreMesh(core_axis_name="c", subcore_axis_name="s")


@functools.partial(
    pl.kernel,
    mesh=_mesh,
    compiler_params=pltpu.CompilerParams(use_tc_tiling_on_sc=True, needs_layout_passes=False),
    out_type=jax.ShapeDtypeStruct((BATCH * 2 * DIM,), jnp.float32),
    scratch_types=[
        pltpu.VMEM((BATCH,), jnp.int32),
        pltpu.VMEM((BATCH + L,), jnp.int32),
        pltpu.VMEM((256,), jnp.int32),
        pltpu.VMEM((256,), jnp.int32),
        pltpu.VMEM((256,), jnp.int32),
        pltpu.VMEM((L + 1,), jnp.int32),
        pltpu.VMEM((RING, DIM, CHU), jnp.float32),
        pltpu.VMEM((RING, DIM, CHU), jnp.float32),
        pltpu.VMEM((SR, 2 * DIM), jnp.float32),
        pltpu.SemaphoreType.DMA,
        pltpu.SemaphoreType.DMA,
    ],
)
def _gather_phase(u_hbm, sT_hbm, bT_hbm, sb_hbm,
                  u_v, wl2, cnts, bases, cur, tmp, s_ch, b_ch, row_ring,
                  sem, sem2):
    wid = lax.axis_index("s") * NC + lax.axis_index("c")
    lane = lax.iota(jnp.int32, L)
    lo = wid * NBPT
    hi = jnp.minimum(lo + NBPT, NBLK)
    nch = hi - lo

    pltpu.sync_copy(u_hbm, u_v)

    def prime(q):
        b = q % RING
        offm = chunk_off_early(q)
        pltpu.async_copy(sT_hbm.at[:, pl.ds(offm, CHU)], s_ch.at[b], sem)
        pltpu.async_copy(bT_hbm.at[:, pl.ds(offm, CHU)], b_ch.at[b], sem)

    def chunk_off_early(q):
        return pl.multiple_of(jnp.minimum((lo + q) * 128, NUSERS - CHU), 128)

    for p in range(RING - 1):
        @pl.when(p < nch)
        def _():
            prime(p)

    for i in range(256 // L):
        cnts[pl.ds(i * L, L)] = jnp.zeros((L,), jnp.int32)

    def count(ci, c):
        uvec = u_v[pl.ds(ci * L, L)]
        blk = uvec >> 7
        m = (blk >= lo) & (blk < hi)
        bs = jnp.minimum(jnp.where(m, blk - lo, 255), 255)
        plsc.addupdate_scatter(cnts, [bs], jnp.where(m, 1, 0), mask=m)
        return c

    lax.fori_loop(0, NVR, count, 0)

    def prefix(i, run):
        cv = cnts[pl.ds(i * L, L)]
        inc = plsc.cumsum(cv)
        ev = run + inc - cv
        bases[pl.ds(i * L, L)] = ev
        cur[pl.ds(i * L, L)] = ev
        return run + jnp.sum(jnp.where(lane == L - 1, inc, 0))

    lax.fori_loop(0, 256 // L, prefix, jnp.int32(0))

    def place(ci, c):
        uvec = u_v[pl.ds(ci * L, L)]
        blk = uvec >> 7
        m = (blk >= lo) & (blk < hi)
        key = jnp.where(m, blk, BIG)
        val = ((ci * L + lane) << 7) | (uvec & 127)
        key_s, val_s = plsc.sort_key_val(key, val)
        m_s = key_s < BIG
        bs = jnp.minimum(jnp.where(m_s, key_s - lo, 255), 255)
        tmp[pl.ds(1, L)] = key_s
        prev = tmp[pl.ds(0, L)]
        boundary = (key_s != prev) | (lane == 0)
        runpos = lane - plsc.cummax(jnp.where(boundary, lane, 0))
        pos = plsc.load_gather(cur, [bs]) + runpos
        plsc.store_scatter(wl2, [pos], val_s, mask=m_s)
        plsc.addupdate_scatter(cur, [bs], jnp.where(m_s, 1, 0), mask=m_s)
        return c

    lax.fori_loop(0, NVR, place, 0)

    def blk_cnt(q):
        cv = cnts[pl.ds((q >> 4) << 4, L)]
        return jnp.sum(jnp.where(lane == (q & 15), cv, 0))

    def blk_base(q):
        bv = bases[pl.ds((q >> 4) << 4, L)]
        return jnp.sum(jnp.where(lane == (q & 15), bv, 0))

    def chunk_off(q):
        return pl.multiple_of(jnp.minimum((lo + q) * 128, NUSERS - CHU), 128)

    def issue(q):
        @pl.when(blk_cnt(q) > 0)
        def _():
            b = q % RING
            offm = chunk_off(q)
            pltpu.async_copy(sT_hbm.at[:, pl.ds(offm, CHU)], s_ch.at[b], sem)
            pltpu.async_copy(bT_hbm.at[:, pl.ds(offm, CHU)], b_ch.at[b], sem)

    def chunk(q, se):
        cq = blk_cnt(q)

        @pl.when(q + RING - 1 < nch)
        def _():
            issue(q + RING - 1)

        @pl.when((q < RING - 1) & (cq == 0))
        def _():
            b = q % RING
            offm = chunk_off(q)
            pltpu.make_async_copy(
                sT_hbm.at[:, pl.ds(offm, CHU)], s_ch.at[b], sem).wait()
            pltpu.make_async_copy(
                bT_hbm.at[:, pl.ds(offm, CHU)], b_ch.at[b], sem).wait()

        def work(se):
            b = q % RING
            offm = chunk_off(q)
            delta = (lo + q) * 128 - offm
            pltpu.make_async_copy(
                sT_hbm.at[:, pl.ds(offm, CHU)], s_ch.at[b], sem).wait()
            pltpu.make_async_copy(
                bT_hbm.at[:, pl.ds(offm, CHU)], b_ch.at[b], sem).wait()
            base_q = blk_base(q)

            def elem(e, se):
                v16 = wl2[pl.ds(base_q + e, L)]
                val = jnp.sum(jnp.where(lane == 0, v16, 0))
                eid = val >> 7
                col = (val & 127) + delta
                cs = jnp.broadcast_to(col, (L,)).astype(jnp.int32)
                slot = se % SR

                @pl.when(se >= SR)
                def _():
                    pltpu.make_async_copy(
                        sb_hbm.at[pl.ds(0, 2 * DIM)], row_ring.at[0], sem2).wait()

                for j in range(DIM // L):
                    rvec = j * L + lane
                    row_ring[slot, pl.ds(j * L, L)] = plsc.load_gather(
                        s_ch.at[b], [rvec, cs])
                    row_ring[slot, pl.ds(DIM + j * L, L)] = plsc.load_gather(
                        b_ch.at[b], [rvec, cs])
                dst = pl.multiple_of(eid * (2 * DIM), 128)
                pltpu.async_copy(
                    row_ring.at[slot], sb_hbm.at[pl.ds(dst, 2 * DIM)], sem2)
                return se + 1

            return lax.fori_loop(0, cq, elem, se)

        return lax.cond(cq > 0, work, lambda t: t, se)

    se = lax.fori_loop(0, nch, chunk, jnp.int32(0))

    def drain(i, c):
        pltpu.make_async_copy(
            sb_hbm.at[pl.ds(0, 2 * DIM)], row_ring.at[0], sem2).wait()
        return c

    lax.fori_loop(0, jnp.minimum(se, SR), drain, 0)


@functools.partial(
    pl.kernel,
    mesh=_mesh,
    compiler_params=pltpu.CompilerParams(use_tc_tiling_on_sc=False),
    out_type=jax.ShapeDtypeStruct((BATCH, DIM), jnp.float32),
    scratch_types=[
        pltpu.VMEM((BPW, DIM), jnp.float32),
        pltpu.VMEM((BPW * 2 * DIM,), jnp.float32),
        pltpu.SemaphoreType.DMA,
    ],
)
def _apply_phase(x_hbm, sb_hbm, out_hbm, x_v, sb_v, sem):
    wid = lax.axis_index("s") * NC + lax.axis_index("c")
    base = wid * BPW
    rows = pl.ds(base, BPW)
    c1 = pltpu.async_copy(sb_hbm.at[pl.ds(base * 2 * DIM, BPW * 2 * DIM)], sb_v, sem)
    pltpu.sync_copy(x_hbm.at[rows], x_v)
    c1.wait()

    @plsc.parallel_loop(0, BPW, 1, unroll=8)
    def row(r):
        for j in range(DIM // L):
            sl = pl.ds(j * L, L)
            x_v[r, sl] = (x_v[r, sl] * sb_v[pl.ds(r * 2 * DIM + j * L, L)]
                          + sb_v[pl.ds(r * 2 * DIM + DIM + j * L, L)])
    pltpu.sync_copy(x_v, out_hbm.at[rows])


def kernel(x, u, scale_weight, bias_weight):
    sb = _gather_phase(u, scale_weight.T, bias_weight.T)
    return _apply_phase(x, sb)


# speedup vs baseline: 1.0020x; 1.0020x over previous
import functools

import jax
import jax.numpy as jnp
from jax import lax
from jax.experimental import pallas as pl
from jax.experimental.pallas import tpu as pltpu
from jax.experimental.pallas import tpu_sc as plsc

BATCH = 16384
DIM = 64
NUSERS = 1000000

_info = plsc.get_sparse_core_info()
NC, NS, L = _info.num_cores, _info.num_subcores, _info.num_lanes
NW = NC * NS
BPW = BATCH // NW
NBLK = (NUSERS + 127) // 128
NBPT = (NBLK + NW - 1) // NW
RING = 5
SR = 8
BIG = 2**31 - 1
CHU = 128
NVR = BATCH // L

_mesh = plsc.VectorSubcoreMesh(core_axis_name="c", subcore_axis_name="s")


@functools.partial(
    pl.kernel,
    mesh=_mesh,
    compiler_params=pltpu.CompilerParams(use_tc_tiling_on_sc=True, needs_layout_passes=False),
    out_type=jax.ShapeDtypeStruct((BATCH * 2 * DIM,), jnp.float32),
    scratch_types=[
        pltpu.VMEM((BATCH,), jnp.int32),
        pltpu.VMEM((BATCH + L,), jnp.int32),
        pltpu.VMEM((256,), jnp.int32),
        pltpu.VMEM((256,), jnp.int32),
        pltpu.VMEM((256,), jnp.int32),
        pltpu.VMEM((L + 1,), jnp.int32),
        pltpu.VMEM((RING, DIM, CHU), jnp.float32),
        pltpu.VMEM((RING, DIM, CHU), jnp.float32),
        pltpu.VMEM((SR, 2 * DIM), jnp.float32),
        pltpu.SemaphoreType.DMA,
        pltpu.SemaphoreType.DMA,
    ],
)
def _gather_phase(u_hbm, sT_hbm, bT_hbm, sb_hbm,
                  u_v, wl2, cnts, bases, cur, tmp, s_ch, b_ch, row_ring,
                  sem, sem2):
    wid = lax.axis_index("s") * NC + lax.axis_index("c")
    lane = lax.iota(jnp.int32, L)
    lo = wid * NBPT
    hi = jnp.minimum(lo + NBPT, NBLK)
    nch = hi - lo

    pltpu.sync_copy(u_hbm, u_v)

    def prime(q):
        b = q % RING
        offm = chunk_off_early(q)
        pltpu.async_copy(sT_hbm.at[:, pl.ds(offm, CHU)], s_ch.at[b], sem)
        pltpu.async_copy(bT_hbm.at[:, pl.ds(offm, CHU)], b_ch.at[b], sem)

    def chunk_off_early(q):
        return pl.multiple_of(jnp.minimum((lo + q) * 128, NUSERS - CHU), 128)

    for p in range(RING - 1):
        @pl.when(p < nch)
        def _():
            prime(p)

    for i in range(256 // L):
        cnts[pl.ds(i * L, L)] = jnp.zeros((L,), jnp.int32)

    def count(ci, c):
        uvec = u_v[pl.ds(ci * L, L)]
        blk = uvec >> 7
        m = (blk >= lo) & (blk < hi)
        bs = jnp.minimum(jnp.where(m, blk - lo, 255), 255)
        plsc.addupdate_scatter(cnts, [bs], jnp.where(m, 1, 0), mask=m)
        return c

    lax.fori_loop(0, NVR, count, 0)

    def prefix(i, run):
        cv = cnts[pl.ds(i * L, L)]
        inc = plsc.cumsum(cv)
        ev = run + inc - cv
        bases[pl.ds(i * L, L)] = ev
        cur[pl.ds(i * L, L)] = ev
        return run + jnp.sum(jnp.where(lane == L - 1, inc, 0))

    lax.fori_loop(0, 256 // L, prefix, jnp.int32(0))

    def place(ci, c):
        uvec = u_v[pl.ds(ci * L, L)]
        blk = uvec >> 7
        m = (blk >= lo) & (blk < hi)
        key = jnp.where(m, blk, BIG)
        val = ((ci * L + lane) << 7) | (uvec & 127)
        key_s, val_s = plsc.sort_key_val(key, val)
        m_s = key_s < BIG
        bs = jnp.minimum(jnp.where(m_s, key_s - lo, 255), 255)
        tmp[pl.ds(1, L)] = key_s
        prev = tmp[pl.ds(0, L)]
        boundary = (key_s != prev) | (lane == 0)
        runpos = lane - plsc.cummax(jnp.where(boundary, lane, 0))
        pos = plsc.load_gather(cur, [bs]) + runpos
        plsc.store_scatter(wl2, [pos], val_s, mask=m_s)
        plsc.addupdate_scatter(cur, [bs], jnp.where(m_s, 1, 0), mask=m_s)
        return c

    lax.fori_loop(0, NVR, place, 0)

    def blk_cnt(q):
        cv = cnts[pl.ds((q >> 4) << 4, L)]
        return jnp.sum(jnp.where(lane == (q & 15), cv, 0))

    def blk_base(q):
        bv = bases[pl.ds((q >> 4) << 4, L)]
        return jnp.sum(jnp.where(lane == (q & 15), bv, 0))

    def chunk_off(q):
        return pl.multiple_of(jnp.minimum((lo + q) * 128, NUSERS - CHU), 128)

    def issue(q):
        @pl.when(blk_cnt(q) > 0)
        def _():
            b = q % RING
            offm = chunk_off(q)
            pltpu.async_copy(sT_hbm.at[:, pl.ds(offm, CHU)], s_ch.at[b], sem)
            pltpu.async_copy(bT_hbm.at[:, pl.ds(offm, CHU)], b_ch.at[b], sem)

    def chunk(q, se):
        cq = blk_cnt(q)

        @pl.when(q + RING - 1 < nch)
        def _():
            issue(q + RING - 1)

        @pl.when((q < RING - 1) & (cq == 0))
        def _():
            b = q % RING
            offm = chunk_off(q)
            pltpu.make_async_copy(
                sT_hbm.at[:, pl.ds(offm, CHU)], s_ch.at[b], sem).wait()
            pltpu.make_async_copy(
                bT_hbm.at[:, pl.ds(offm, CHU)], b_ch.at[b], sem).wait()

        def work(se):
            b = q % RING
            offm = chunk_off(q)
            delta = (lo + q) * 128 - offm
            pltpu.make_async_copy(
                sT_hbm.at[:, pl.ds(offm, CHU)], s_ch.at[b], sem).wait()
            pltpu.make_async_copy(
                bT_hbm.at[:, pl.ds(offm, CHU)], b_ch.at[b], sem).wait()
            base_q = blk_base(q)

            def elem(e, se):
                v16 = wl2[pl.ds(base_q + e, L)]
                val = jnp.sum(jnp.where(lane == 0, v16, 0))
                eid = val >> 7
                col = (val & 127) + delta
                cs = jnp.broadcast_to(col, (L,)).astype(jnp.int32)
                slot = se % SR

                @pl.when(se >= SR)
                def _():
                    pltpu.make_async_copy(
                        sb_hbm.at[pl.ds(0, 2 * DIM)], row_ring.at[0], sem2).wait()

                for j in range(DIM // L):
                    rvec = j * L + lane
                    row_ring[slot, pl.ds(j * L, L)] = plsc.load_gather(
                        s_ch.at[b], [rvec, cs])
                    row_ring[slot, pl.ds(DIM + j * L, L)] = plsc.load_gather(
                        b_ch.at[b], [rvec, cs])
                dst = pl.multiple_of(eid * (2 * DIM), 128)
                pltpu.async_copy(
                    row_ring.at[slot], sb_hbm.at[pl.ds(dst, 2 * DIM)], sem2)
                return se + 1

            return lax.fori_loop(0, cq, elem, se)

        return lax.cond(cq > 0, work, lambda t: t, se)

    se = lax.fori_loop(0, nch, chunk, jnp.int32(0))

    def drain(i, c):
        pltpu.make_async_copy(
            sb_hbm.at[pl.ds(0, 2 * DIM)], row_ring.at[0], sem2).wait()
        return c

    lax.fori_loop(0, jnp.minimum(se, SR), drain, 0)


@functools.partial(
    pl.kernel,
    mesh=_mesh,
    compiler_params=pltpu.CompilerParams(use_tc_tiling_on_sc=False),
    out_type=jax.ShapeDtypeStruct((BATCH, DIM), jnp.float32),
    scratch_types=[
        pltpu.VMEM((BPW, DIM), jnp.float32),
        pltpu.VMEM((BPW * 2 * DIM,), jnp.float32),
        pltpu.SemaphoreType.DMA,
    ],
)
def _apply_phase(x_hbm, sb_hbm, out_hbm, x_v, sb_v, sem):
    wid = lax.axis_index("s") * NC + lax.axis_index("c")
    base = wid * BPW
    rows = pl.ds(base, BPW)
    c1 = pltpu.async_copy(sb_hbm.at[pl.ds(base * 2 * DIM, BPW * 2 * DIM)], sb_v, sem)
    pltpu.sync_copy(x_hbm.at[rows], x_v)
    c1.wait()

    @plsc.parallel_loop(0, BPW, 1, unroll=4)
    def row(r):
        for j in range(DIM // L):
            sl = pl.ds(j * L, L)
            x_v[r, sl] = (x_v[r, sl] * sb_v[pl.ds(r * 2 * DIM + j * L, L)]
                          + sb_v[pl.ds(r * 2 * DIM + DIM + j * L, L)])
    pltpu.sync_copy(x_v, out_hbm.at[rows])


def kernel(x, u, scale_weight, bias_weight):
    sb = _gather_phase(u, scale_weight.T, bias_weight.T)
    return _apply_phase(x, sb)
